# Initial kernel scaffold; baseline (speedup 1.0000x reference)
import jax
import jax.numpy as jnp
from jax.experimental import pallas as pl


def kernel(ids, weight):
    raise NotImplementedError("write your pallas kernel here")


# speedup vs baseline: 2.9621x; 2.9621x over previous
import functools

import jax
import jax.numpy as jnp
from jax import lax
from jax.experimental import pallas as pl
from jax.experimental.pallas import tpu as pltpu
from jax.experimental.pallas import tpu_sc as plsc

_NUM_WORKERS = 32
_CHUNK = 128


@functools.lru_cache(maxsize=None)
def _build(B, V, D):
  b_per_w = B // _NUM_WORKERS
  n_chunks = b_per_w // _CHUNK
  mesh = plsc.VectorSubcoreMesh(core_axis_name="c", subcore_axis_name="s")

  @functools.partial(
      pl.kernel,
      out_type=jax.ShapeDtypeStruct((B, D), jnp.float32),
      mesh=mesh,
      scratch_types=[
          pltpu.VMEM((n_chunks, _CHUNK), jnp.int32),
          pltpu.VMEM((_CHUNK, D), jnp.float32),
          pltpu.SemaphoreType.DMA,
      ],
  )
  def emb(ids_hbm, w_hbm, out_hbm, idx_v, rows_v, gsem):
    cid = lax.axis_index("c")
    sid = lax.axis_index("s")
    wid = sid * 2 + cid
    base = wid * b_per_w
    pltpu.sync_copy(ids_hbm.at[wid], idx_v)

    def body(j, _):
      pltpu.async_copy(w_hbm.at[idx_v.at[j]], rows_v, gsem).wait()
      pltpu.sync_copy(rows_v, out_hbm.at[pl.ds(base + j * _CHUNK, _CHUNK)])
      return 0

    lax.fori_loop(0, n_chunks, body, 0)

  return emb


def kernel(ids, weight):
  Bt, H = ids.shape
  V, D = weight.shape
  B = Bt * H
  flat = ids.astype(jnp.int32).reshape(
      _NUM_WORKERS, B // _NUM_WORKERS // _CHUNK, _CHUNK)
  out = _build(B, V, D)(flat, weight)
  return out.reshape(Bt, H, D)


# speedup vs baseline: 3.3134x; 1.1186x over previous
import functools

import jax
import jax.numpy as jnp
from jax import lax
from jax.experimental import pallas as pl
from jax.experimental.pallas import tpu as pltpu
from jax.experimental.pallas import tpu_sc as plsc

_NUM_WORKERS = 32
_CHUNK = 128
_NBUF = 5


@functools.lru_cache(maxsize=None)
def _build(B, V, D):
  b_per_w = B // _NUM_WORKERS
  n_chunks = b_per_w // _CHUNK
  n_outer = n_chunks // _NBUF
  mesh = plsc.VectorSubcoreMesh(core_axis_name="c", subcore_axis_name="s")

  @functools.partial(
      pl.kernel,
      out_type=jax.ShapeDtypeStruct((B, D), jnp.float32),
      mesh=mesh,
      scratch_types=[
          pltpu.VMEM((n_chunks, _CHUNK), jnp.int32),
          pltpu.VMEM((_NBUF, _CHUNK, D), jnp.float32),
          pltpu.SemaphoreType.DMA((_NBUF,)),
          pltpu.SemaphoreType.DMA((_NBUF,)),
      ],
  )
  def emb(ids_hbm, w_hbm, out_hbm, idx_v, rows_v, gsem, ssem):
    cid = lax.axis_index("c")
    sid = lax.axis_index("s")
    wid = sid * 2 + cid
    base = wid * b_per_w
    pltpu.sync_copy(ids_hbm.at[wid], idx_v)

    def drain_gather(b):
      pltpu.make_async_copy(
          w_hbm.at[pl.ds(0, _CHUNK)], rows_v.at[b], gsem.at[b]).wait()

    def drain_write(b):
      pltpu.make_async_copy(
          rows_v.at[b], out_hbm.at[pl.ds(base, _CHUNK)], ssem.at[b]).wait()

    for b in range(_NBUF):
      pltpu.async_copy(w_hbm.at[idx_v.at[b]], rows_v.at[b], gsem.at[b])

    def outer(o, _):
      jo = o * _NBUF
      for b in range(_NBUF):
        drain_gather(b)
        pltpu.async_copy(
            rows_v.at[b],
            out_hbm.at[pl.ds(base + (jo + b) * _CHUNK, _CHUNK)],
            ssem.at[b])
      @pl.when(o + 1 < n_outer)
      def _():
        for b in range(_NBUF):
          drain_write(b)
          pltpu.async_copy(
              w_hbm.at[idx_v.at[jo + _NBUF + b]], rows_v.at[b], gsem.at[b])
      return 0

    lax.fori_loop(0, n_outer, outer, 0)
    for b in range(_NBUF):
      drain_write(b)

  return emb


def kernel(ids, weight):
  Bt, H = ids.shape
  V, D = weight.shape
  B = Bt * H
  flat = ids.astype(jnp.int32).reshape(
      _NUM_WORKERS, B // _NUM_WORKERS // _CHUNK, _CHUNK)
  out = _build(B, V, D)(flat, weight)
  return out.reshape(Bt, H, D)
